# baseline (speedup 1.0000x reference)
import functools

import jax
import jax.numpy as jnp
from jax import lax
from jax.experimental import pallas as pl
from jax.experimental.pallas import tpu as pltpu
from jax.experimental.pallas import tpu_sc as plsc

BATCH = 4096
HIST = 200
EMBED_DIM = 64

NUM_CORES = 2
NUM_SUBCORES = 16
NUM_WORKERS = NUM_CORES * NUM_SUBCORES

N = BATCH * HIST
PER_WORKER = N // NUM_WORKERS
CHUNK = 512
NUM_CHUNKS = PER_WORKER // CHUNK

_mesh = plsc.VectorSubcoreMesh(core_axis_name="c", subcore_axis_name="s")


@functools.partial(
    pl.kernel,
    mesh=_mesh,
    out_type=jax.ShapeDtypeStruct((N, EMBED_DIM), jnp.float32),
    scratch_types=[
        pltpu.VMEM((CHUNK,), jnp.int32),
        pltpu.VMEM((CHUNK, EMBED_DIM), jnp.float32),
        pltpu.SemaphoreType.DMA,
    ],
    compiler_params=pltpu.CompilerParams(use_tc_tiling_on_sc=False),
)
def _sc_gather(idx_hbm, table_hbm, out_hbm, idx_v, rows_v, sem):
    wid = lax.axis_index("s") * NUM_CORES + lax.axis_index("c")
    base = wid * PER_WORKER

    def body(i, carry):
        off = base + i * CHUNK
        pltpu.sync_copy(idx_hbm.at[pl.ds(off, CHUNK)], idx_v)
        pltpu.async_copy(table_hbm.at[idx_v], rows_v, sem).wait()
        pltpu.sync_copy(rows_v, out_hbm.at[pl.ds(off, CHUNK)])
        return carry

    lax.fori_loop(0, NUM_CHUNKS, body, 0)


def kernel(x, table):
    idx = x.reshape(-1).astype(jnp.int32)
    out = _sc_gather(idx, table)
    return out.reshape(x.shape + (table.shape[1],))


# speedup vs baseline: 1.0413x; 1.0413x over previous
import functools

import jax
import jax.numpy as jnp
from jax import lax
from jax.experimental import pallas as pl
from jax.experimental.pallas import tpu as pltpu
from jax.experimental.pallas import tpu_sc as plsc

BATCH = 4096
HIST = 200
EMBED_DIM = 64

NUM_CORES = 2
NUM_SUBCORES = 16
NUM_WORKERS = NUM_CORES * NUM_SUBCORES

N = BATCH * HIST
PER_WORKER = N // NUM_WORKERS
CHUNK = 256
NBUF = 4
NUM_CHUNKS = PER_WORKER // CHUNK
NUM_GROUPS = NUM_CHUNKS // NBUF

_mesh = plsc.VectorSubcoreMesh(core_axis_name="c", subcore_axis_name="s")


@functools.partial(
    pl.kernel,
    mesh=_mesh,
    out_type=jax.ShapeDtypeStruct((N, EMBED_DIM), jnp.float32),
    scratch_types=[
        pltpu.VMEM((PER_WORKER,), jnp.int32),
        [pltpu.VMEM((CHUNK, EMBED_DIM), jnp.float32) for _ in range(NBUF)],
        [pltpu.SemaphoreType.DMA for _ in range(NBUF)],
        [pltpu.SemaphoreType.DMA for _ in range(NBUF)],
    ],
    compiler_params=pltpu.CompilerParams(use_tc_tiling_on_sc=False),
)
def _sc_gather(idx_hbm, table_hbm, out_hbm, idx_v, rows, gsems, osems):
    wid = lax.axis_index("s") * NUM_CORES + lax.axis_index("c")
    base = wid * PER_WORKER

    pltpu.sync_copy(idx_hbm.at[pl.ds(base, PER_WORKER)], idx_v)

    def gather_start(chunk_i, k):
        pltpu.async_copy(
            table_hbm.at[idx_v.at[pl.ds(chunk_i * CHUNK, CHUNK)]],
            rows[k], gsems[k])

    def gather_wait(k):
        pltpu.make_async_copy(
            table_hbm.at[idx_v.at[pl.ds(0, CHUNK)]],
            rows[k], gsems[k]).wait()

    def wb_start(chunk_i, k):
        pltpu.async_copy(
            rows[k], out_hbm.at[pl.ds(base + chunk_i * CHUNK, CHUNK)],
            osems[k])

    def wb_wait(k):
        pltpu.make_async_copy(
            rows[k], out_hbm.at[pl.ds(base, CHUNK)], osems[k]).wait()

    for k in range(NBUF):
        gather_start(k, k)

    def body(g, carry):
        c0 = g * NBUF
        for k in range(NBUF):
            gather_wait(k)
            wb_start(c0 + k, k)

        @pl.when(g < NUM_GROUPS - 1)
        def _():
            for k in range(NBUF):
                wb_wait(k)
                gather_start(c0 + NBUF + k, k)

        return carry

    lax.fori_loop(0, NUM_GROUPS, body, 0)

    for k in range(NBUF):
        wb_wait(k)


def kernel(x, table):
    idx = x.reshape(-1).astype(jnp.int32)
    out = _sc_gather(idx, table)
    return out.reshape(x.shape + (table.shape[1],))


# speedup vs baseline: 1.2696x; 1.2192x over previous
import functools

import jax
import jax.numpy as jnp
from jax import lax
from jax.experimental import pallas as pl
from jax.experimental.pallas import tpu as pltpu
from jax.experimental.pallas import tpu_sc as plsc

BATCH = 4096
HIST = 200
EMBED_DIM = 64
SLOT = 128

NUM_CORES = 2
NUM_SUBCORES = 16
NUM_WORKERS = NUM_CORES * NUM_SUBCORES

N = BATCH * HIST
PER_WORKER = N // NUM_WORKERS
CHUNK = 128
NBUF = 4
NUM_CHUNKS = PER_WORKER // CHUNK
NUM_GROUPS = NUM_CHUNKS // NBUF

_mesh = plsc.VectorSubcoreMesh(core_axis_name="c", subcore_axis_name="s")


@functools.partial(
    pl.kernel,
    mesh=_mesh,
    out_type=jax.ShapeDtypeStruct((N, SLOT), jnp.float32),
    scratch_types=[
        pltpu.VMEM((PER_WORKER,), jnp.int32),
        [pltpu.VMEM((CHUNK, SLOT), jnp.float32) for _ in range(NBUF)],
        [pltpu.SemaphoreType.DMA for _ in range(NBUF)],
        [pltpu.SemaphoreType.DMA for _ in range(NBUF)],
    ],
    compiler_params=pltpu.CompilerParams(use_tc_tiling_on_sc=True),
)
def _sc_gather(idx_hbm, table_hbm, out_hbm, idx_v, rows, gsems, osems):
    wid = lax.axis_index("s") * NUM_CORES + lax.axis_index("c")
    base = wid * PER_WORKER

    pltpu.sync_copy(idx_hbm.at[pl.ds(base, PER_WORKER)], idx_v)

    def gather_start(chunk_i, k):
        pltpu.async_copy(
            table_hbm.at[idx_v.at[pl.ds(chunk_i * CHUNK, CHUNK)]],
            rows[k], gsems[k])

    def gather_wait(k):
        pltpu.make_async_copy(
            table_hbm.at[idx_v.at[pl.ds(0, CHUNK)]],
            rows[k], gsems[k]).wait()

    def wb_start(chunk_i, k):
        pltpu.async_copy(
            rows[k], out_hbm.at[pl.ds(base + chunk_i * CHUNK, CHUNK)],
            osems[k])

    def wb_wait(k):
        pltpu.make_async_copy(
            rows[k], out_hbm.at[pl.ds(base, CHUNK)], osems[k]).wait()

    for k in range(NBUF):
        gather_start(k, k)

    def body(g, carry):
        c0 = g * NBUF
        for k in range(NBUF):
            gather_wait(k)
            wb_start(c0 + k, k)

        @pl.when(g < NUM_GROUPS - 1)
        def _():
            for k in range(NBUF):
                wb_wait(k)
                gather_start(c0 + NBUF + k, k)

        return carry

    lax.fori_loop(0, NUM_GROUPS, body, 0)

    for k in range(NBUF):
        wb_wait(k)


def kernel(x, table):
    tpad = jnp.pad(table, ((0, 0), (0, SLOT - EMBED_DIM)))
    idx = x.reshape(-1).astype(jnp.int32)
    out128 = _sc_gather(idx, tpad)
    return out128[:, :EMBED_DIM].reshape(x.shape + (table.shape[1],))


# speedup vs baseline: 1.3856x; 1.0914x over previous
import functools

import jax
import jax.numpy as jnp
from jax import lax
from jax.experimental import pallas as pl
from jax.experimental.pallas import tpu as pltpu
from jax.experimental.pallas import tpu_sc as plsc

BATCH = 4096
HIST = 200
EMBED_DIM = 64
SLOT = 128

NUM_CORES = 2
NUM_SUBCORES = 16
NUM_WORKERS = NUM_CORES * NUM_SUBCORES

N = BATCH * HIST
PER_WORKER = N // NUM_WORKERS
CHUNK = 256
NBUF = 4
NUM_CHUNKS = PER_WORKER // CHUNK
NUM_GROUPS = NUM_CHUNKS // NBUF

_mesh = plsc.VectorSubcoreMesh(core_axis_name="c", subcore_axis_name="s")


@functools.partial(
    pl.kernel,
    mesh=_mesh,
    out_type=jax.ShapeDtypeStruct((N, SLOT), jnp.float32),
    scratch_types=[
        pltpu.VMEM((PER_WORKER,), jnp.int32),
        [pltpu.VMEM((CHUNK, EMBED_DIM), jnp.float32) for _ in range(NBUF)],
        [pltpu.SemaphoreType.DMA for _ in range(NBUF)],
        [pltpu.SemaphoreType.DMA for _ in range(NBUF)],
    ],
    compiler_params=pltpu.CompilerParams(use_tc_tiling_on_sc=False),
)
def _sc_gather(idx_hbm, table_hbm, out_hbm, idx_v, rows, gsems, osems):
    wid = lax.axis_index("s") * NUM_CORES + lax.axis_index("c")
    base = wid * PER_WORKER

    pltpu.sync_copy(idx_hbm.at[pl.ds(base, PER_WORKER)], idx_v)

    def gather_start(chunk_i, k):
        pltpu.async_copy(
            table_hbm.at[idx_v.at[pl.ds(chunk_i * CHUNK, CHUNK)]],
            rows[k], gsems[k])

    def gather_wait(k):
        pltpu.make_async_copy(
            table_hbm.at[idx_v.at[pl.ds(0, CHUNK)]],
            rows[k], gsems[k]).wait()

    def wb_start(chunk_i, k):
        pltpu.async_copy(
            rows[k],
            out_hbm.at[pl.ds(base + chunk_i * CHUNK, CHUNK),
                       pl.ds(0, EMBED_DIM)],
            osems[k])

    def wb_wait(k):
        pltpu.make_async_copy(
            rows[k],
            out_hbm.at[pl.ds(base, CHUNK), pl.ds(0, EMBED_DIM)],
            osems[k]).wait()

    for k in range(NBUF):
        gather_start(k, k)

    def body(g, carry):
        c0 = g * NBUF
        for k in range(NBUF):
            gather_wait(k)
            wb_start(c0 + k, k)

        @pl.when(g < NUM_GROUPS - 1)
        def _():
            for k in range(NBUF):
                wb_wait(k)
                gather_start(c0 + NBUF + k, k)

        return carry

    lax.fori_loop(0, NUM_GROUPS, body, 0)

    for k in range(NBUF):
        wb_wait(k)


def kernel(x, table):
    idx = x.reshape(-1).astype(jnp.int32)
    out128 = _sc_gather(idx, table)
    return out128[:, :EMBED_DIM].reshape(x.shape + (table.shape[1],))
